# baseline (speedup 1.0000x reference)
import functools

import jax
import jax.numpy as jnp
from jax import lax
from jax.experimental import pallas as pl
from jax.experimental.pallas import tpu as pltpu
from jax.experimental.pallas import tpu_sc as plsc


def _lrelu(x):
    return jnp.where(x >= 0, x, 0.01 * x)


def _l2rows(x):
    return x / jnp.maximum(jnp.sqrt(jnp.sum(x * x, axis=1, keepdims=True)), 1e-12)


def _sc_gather2(table1, table2, idx):
    C = idx.shape[0]
    D = table1.shape[1]
    info = plsc.get_sparse_core_info()
    nw = info.num_cores * info.num_subcores
    half = nw // 2
    b = C // half
    mesh = plsc.VectorSubcoreMesh(core_axis_name="c", subcore_axis_name="s")

    @functools.partial(
        pl.kernel,
        mesh=mesh,
        out_type=jax.ShapeDtypeStruct((2 * C, D), jnp.float32),
        scratch_types=[
            pltpu.VMEM((b,), jnp.int32),
            pltpu.VMEM((b, D), jnp.float32),
            pltpu.SemaphoreType.DMA,
        ],
    )
    def k(t1_hbm, t2_hbm, idx_hbm, out_hbm, idx_v, rows_v, sem):
        wid = lax.axis_index("s") * info.num_cores + lax.axis_index("c")
        slot = wid % half
        base = slot * b
        pltpu.sync_copy(idx_hbm.at[pl.ds(base, b)], idx_v)

        @pl.when(wid < half)
        def _():
            pltpu.async_copy(t1_hbm.at[idx_v], rows_v, sem).wait()
            pltpu.sync_copy(rows_v, out_hbm.at[pl.ds(base, b)])

        @pl.when(wid >= half)
        def _():
            pltpu.async_copy(t2_hbm.at[idx_v], rows_v, sem).wait()
            pltpu.sync_copy(rows_v, out_hbm.at[pl.ds(C + base, b)])

    return k(table1, table2, idx)


def _agg2_body(a_ref, nf_ref, w1_ref, w2_ref, da1_ref, da2_ref, abfo_ref,
               diag_ref, abf_ref, xw1_ref, xw2_ref, *, bm):
    p = pl.program_id(0)
    i = pl.program_id(1)

    @pl.when((p == 0) & (i == 0))
    def _():
        xw1_ref[...] = jnp.dot(nf_ref[...], w1_ref[...],
                               preferred_element_type=jnp.float32)

    @pl.when(p == 0)
    def _():
        a = a_ref[...]
        abf_ref[pl.ds(i * bm, bm), :] = a.astype(jnp.bfloat16)
        r = lax.broadcasted_iota(jnp.int32, a.shape, 0)
        c = lax.broadcasted_iota(jnp.int32, a.shape, 1)
        diag_ref[...] = jnp.sum(jnp.where(c == r + i * bm, a, 0.0),
                                axis=1)[None, :]
        da1 = _lrelu(jnp.dot(a, xw1_ref[...],
                             preferred_element_type=jnp.float32))
        da1_ref[...] = da1
        xw2_ref[pl.ds(i * bm, bm), :] = jnp.dot(
            _lrelu(da1), w2_ref[...],
            preferred_element_type=jnp.float32).astype(jnp.bfloat16)

    @pl.when(p == 1)
    def _():
        ab = abf_ref[pl.ds(i * bm, bm), :]
        abfo_ref[...] = ab
        da2_ref[...] = _lrelu(jnp.dot(ab, xw2_ref[...],
                                      preferred_element_type=jnp.float32))


def _gcn_two_layers(adj, nf, w1, w2, bm=256):
    m = adj.shape[0]
    n = w1.shape[1]
    kin = nf.shape[1]
    ni = m // bm
    last = ni - 1
    return pl.pallas_call(
        functools.partial(_agg2_body, bm=bm),
        grid=(2, ni),
        in_specs=[
            pl.BlockSpec((bm, m), lambda p, i: ((1 - p) * i + p * last, 0)),
            pl.BlockSpec((m, kin), lambda p, i: (0, 0)),
            pl.BlockSpec((kin, n), lambda p, i: (0, 0)),
            pl.BlockSpec((n, n), lambda p, i: (0, 0)),
        ],
        out_specs=(
            pl.BlockSpec((bm, n), lambda p, i: ((1 - p) * i + p * last, 0)),
            pl.BlockSpec((bm, n), lambda p, i: (i * p, 0)),
            pl.BlockSpec((bm, m), lambda p, i: (i * p, 0)),
            pl.BlockSpec((1, bm), lambda p, i: (0, (1 - p) * i + p * last)),
        ),
        out_shape=(
            jax.ShapeDtypeStruct((m, n), jnp.float32),
            jax.ShapeDtypeStruct((m, n), jnp.float32),
            jax.ShapeDtypeStruct((m, m), jnp.bfloat16),
            jax.ShapeDtypeStruct((1, m), jnp.float32),
        ),
        scratch_shapes=[
            pltpu.VMEM((m, m), jnp.bfloat16),
            pltpu.VMEM((m, n), jnp.float32),
            pltpu.VMEM((m, n), jnp.bfloat16),
        ],
        compiler_params=pltpu.CompilerParams(
            dimension_semantics=("arbitrary", "arbitrary")),
    )(adj, nf, w1, w2)


def _qtower_compute(qa_ref, nfq_ref, w1_ref, w2_ref, g_ref, att_ref, emb_ref,
                    *, C):
    qa = qa_ref[...]
    q1 = _lrelu(jnp.dot(qa, jnp.dot(nfq_ref[...], w1_ref[...],
                                    preferred_element_type=jnp.float32),
                        preferred_element_type=jnp.float32))
    g1 = g_ref[:C, :]
    c1 = lax.dot_general(_l2rows(q1), _l2rows(g1), (((1,), (1,)), ((), ())),
                         preferred_element_type=jnp.float32)
    h1 = jnp.dot(c1, g1, preferred_element_type=jnp.float32)
    h1n = h1 / jnp.maximum(
        jnp.sqrt(jnp.sum(h1 * h1, axis=0, keepdims=True)), 1e-12)
    att_q1 = _lrelu(q1 + h1n)
    q2 = _lrelu(jnp.dot(qa, jnp.dot(att_q1, w2_ref[...],
                                    preferred_element_type=jnp.float32),
                        preferred_element_type=jnp.float32))
    g2 = g_ref[C:, :]
    c2 = lax.dot_general(_l2rows(q2), _l2rows(g2), (((1,), (1,)), ((), ())),
                         preferred_element_type=jnp.float32)
    h2 = jnp.dot(c2, g2, preferred_element_type=jnp.float32)
    h2n = h2 / jnp.maximum(
        jnp.sqrt(jnp.sum(h2 * h2, axis=0, keepdims=True)), 1e-12)
    att = _lrelu(q2 + h2n)
    att_ref[...] = att
    emb_ref[...] = jnp.sum(att, axis=0, keepdims=True) / q2.shape[0]


def _maskstats_body(a_ref, da2_ref, diag_ref, qa_ref, nfq_ref, w1q_ref,
                    w2q_ref, g12_ref, thr_ref, att_ref, end_ref, out_ref,
                    attq_ref, fm_ref, fmb_ref, g_ref, msk_ref, emb_ref,
                    *, bm, C):
    p = pl.program_id(0)
    i = pl.program_id(1)

    @pl.when((p == 0) & (i == 0))
    def _():
        _qtower_compute(qa_ref, nfq_ref, w1q_ref, w2q_ref, g12_ref,
                        attq_ref, emb_ref, C=C)

    @pl.when(p == 0)
    def _():
        att = _lrelu(da2_ref[...])
        att_ref[...] = att
        emb = emb_ref[...]
        emb_norm = jnp.sqrt(jnp.sum(emb * emb))
        row_norm = jnp.sqrt(jnp.sum(att * att, axis=1))
        num = jnp.sum(att * emb, axis=1)
        den = jnp.maximum(emb_norm * row_norm, 1e-8)
        endv = num / den
        end_ref[...] = endv[None, :]
        maskv = (endv > thr_ref[0]).astype(jnp.float32)
        fm = att * maskv[:, None]
        fm_ref[pl.ds(i * bm, bm), :] = fm
        fmb_ref[pl.ds(i * bm, bm), :] = fm.astype(jnp.bfloat16)
        msk_ref[pl.ds(i, 1), :] = maskv[None, :]

        @pl.when(i == 0)
        def _():
            g_ref[...] = jnp.zeros_like(g_ref)

        g_ref[...] += lax.dot_general(fm, fm, (((0,), (0,)), ((), ())),
                                      preferred_element_type=jnp.float32)

    @pl.when(p == 1)
    def _():
        a = a_ref[...]
        fmi = fm_ref[pl.ds(i * bm, bm), :]
        tfm = jnp.dot(a, fmb_ref[...], preferred_element_type=jnp.float32)
        fmg = jnp.dot(fmi, g_ref[...], preferred_element_type=jnp.float32)
        qf = jnp.maximum(jnp.sum(fmi * fmg, axis=1), 0.0)
        inv = 1.0 / jnp.maximum(jnp.sqrt(qf), 1e-12)
        rowdot = jnp.sum(fmi * tfm, axis=1)
        total_c = jnp.sum(rowdot * inv)
        fnorm2 = jnp.sum(fmi * fmi, axis=1)
        diag = diag_ref[0, pl.ds(i * bm, bm)]
        tr_c = jnp.sum(fnorm2 * diag * inv)
        cnt_c = jnp.sum(msk_ref[pl.ds(i, 1), :])
        lane = lax.broadcasted_iota(jnp.int32, (1, 1, 128), 2)
        out_ref[...] = jnp.where(
            lane == 0, total_c,
            jnp.where(lane == 1, tr_c, jnp.where(lane == 2, cnt_c, 0.0)))


def _mask_stats_stage(abf, da2, diag, query_adj, nf_q, w1_q, w2_q, g12, thr,
                      bm=1024):
    m, n = da2.shape
    nq = query_adj.shape[0]
    kq = nf_q.shape[1]
    C = g12.shape[0] // 2
    ni = m // bm
    last = ni - 1
    return pl.pallas_call(
        functools.partial(_maskstats_body, bm=bm, C=C),
        grid=(2, ni),
        in_specs=[
            pl.BlockSpec((bm, m), lambda p, i: (i * p, 0)),
            pl.BlockSpec((bm, n), lambda p, i: ((1 - p) * i + p * last, 0)),
            pl.BlockSpec((1, m), lambda p, i: (0, 0)),
            pl.BlockSpec((nq, nq), lambda p, i: (0, 0)),
            pl.BlockSpec((nq, kq), lambda p, i: (0, 0)),
            pl.BlockSpec((kq, n), lambda p, i: (0, 0)),
            pl.BlockSpec((n, n), lambda p, i: (0, 0)),
            pl.BlockSpec((2 * C, n), lambda p, i: (0, 0)),
            pl.BlockSpec(memory_space=pltpu.SMEM),
        ],
        out_specs=(
            pl.BlockSpec((bm, n), lambda p, i: ((1 - p) * i + p * last, 0)),
            pl.BlockSpec((1, bm), lambda p, i: (0, (1 - p) * i + p * last)),
            pl.BlockSpec((1, 1, 128), lambda p, i: (i * p, 0, 0)),
            pl.BlockSpec((nq, n), lambda p, i: (0, 0)),
        ),
        out_shape=(
            jax.ShapeDtypeStruct((m, n), jnp.float32),
            jax.ShapeDtypeStruct((1, m), jnp.float32),
            jax.ShapeDtypeStruct((ni, 1, 128), jnp.float32),
            jax.ShapeDtypeStruct((nq, n), jnp.float32),
        ),
        scratch_shapes=[
            pltpu.VMEM((m, n), jnp.float32),
            pltpu.VMEM((m, n), jnp.bfloat16),
            pltpu.VMEM((n, n), jnp.float32),
            pltpu.VMEM((ni, bm), jnp.float32),
            pltpu.VMEM((1, n), jnp.float32),
        ],
        compiler_params=pltpu.CompilerParams(
            dimension_semantics=("arbitrary", "arbitrary")),
    )(abf, da2, diag, query_adj, nf_q, w1_q, w2_q, g12, thr)


def kernel(target_adj, node_features_da, query_adj, node_features_q,
           candidate_set, candidate_adj, threshold, W1_da, W1_q, W2_da,
           W2_q):
    del candidate_adj

    da1, da2, abf, adiag = _gcn_two_layers(target_adj, node_features_da,
                                           W1_da, W2_da)

    g12 = _sc_gather2(da1, da2, candidate_set)

    thr = jnp.reshape(threshold.astype(jnp.float32), (1,))
    att_da2, end, stats, att_q2 = _mask_stats_stage(
        abf, da2, adiag, query_adj, node_features_q, W1_q, W2_q, g12, thr)

    total = jnp.sum(stats[:, 0, 0])
    tr = jnp.sum(stats[:, 0, 1])
    cnt = jnp.sum(stats[:, 0, 2])
    pre_avg_degree = jnp.where(cnt > 0, total / jnp.maximum(cnt, 1.0), 0.0)
    pre_density = jnp.where(cnt > 0,
                            2.0 * total / (tr * (tr - 1.0) + 1e-4), 0.0)
    pre_avg_nodes = jnp.where(cnt > 0, tr, 0.0)
    return end, att_da2, att_q2, pre_avg_degree, pre_density, pre_avg_nodes


# speedup vs baseline: 1.0240x; 1.0240x over previous
import functools

import jax
import jax.numpy as jnp
from jax import lax
from jax.experimental import pallas as pl
from jax.experimental.pallas import tpu as pltpu
from jax.experimental.pallas import tpu_sc as plsc


def _lrelu(x):
    return jnp.where(x >= 0, x, 0.01 * x)


def _l2rows(x):
    return x / jnp.maximum(jnp.sqrt(jnp.sum(x * x, axis=1, keepdims=True)), 1e-12)


def _sc_gather2(table1, table2, idx):
    C = idx.shape[0]
    D = table1.shape[1]
    info = plsc.get_sparse_core_info()
    nw = info.num_cores * info.num_subcores
    half = nw // 2
    b = C // half
    mesh = plsc.VectorSubcoreMesh(core_axis_name="c", subcore_axis_name="s")

    @functools.partial(
        pl.kernel,
        mesh=mesh,
        out_type=jax.ShapeDtypeStruct((2 * C, D), jnp.float32),
        scratch_types=[
            pltpu.VMEM((b,), jnp.int32),
            pltpu.VMEM((b, D), jnp.float32),
            pltpu.SemaphoreType.DMA,
        ],
    )
    def k(t1_hbm, t2_hbm, idx_hbm, out_hbm, idx_v, rows_v, sem):
        wid = lax.axis_index("s") * info.num_cores + lax.axis_index("c")
        slot = wid % half
        base = slot * b
        pltpu.sync_copy(idx_hbm.at[pl.ds(base, b)], idx_v)

        @pl.when(wid < half)
        def _():
            pltpu.async_copy(t1_hbm.at[idx_v], rows_v, sem).wait()
            pltpu.sync_copy(rows_v, out_hbm.at[pl.ds(base, b)])

        @pl.when(wid >= half)
        def _():
            pltpu.async_copy(t2_hbm.at[idx_v], rows_v, sem).wait()
            pltpu.sync_copy(rows_v, out_hbm.at[pl.ds(C + base, b)])

    return k(table1, table2, idx)


def _agg2_body(a_ref, nf_ref, w1_ref, w2_ref, da1_ref, da2_ref, abfo_ref,
               diag_ref, abf_ref, xw1_ref, xw2_ref, *, bm):
    p = pl.program_id(0)
    i = pl.program_id(1)

    @pl.when((p == 0) & (i == 0))
    def _():
        xw1_ref[...] = jnp.dot(nf_ref[...], w1_ref[...],
                               preferred_element_type=jnp.float32)

    @pl.when(p == 0)
    def _():
        a = a_ref[...]
        abf_ref[pl.ds(i * bm, bm), :] = a.astype(jnp.bfloat16)
        sub = a_ref[:, pl.ds(i * bm, bm)]
        r = lax.broadcasted_iota(jnp.int32, (bm, bm), 0)
        c = lax.broadcasted_iota(jnp.int32, (bm, bm), 1)
        diag_ref[...] = jnp.sum(jnp.where(c == r, sub, 0.0), axis=1)[None, :]
        da1 = _lrelu(jnp.dot(a, xw1_ref[...],
                             preferred_element_type=jnp.float32))
        da1_ref[...] = da1
        xw2_ref[pl.ds(i * bm, bm), :] = jnp.dot(
            _lrelu(da1), w2_ref[...],
            preferred_element_type=jnp.float32).astype(jnp.bfloat16)

    @pl.when(p == 1)
    def _():
        ab = abf_ref[pl.ds(i * bm, bm), :]
        abfo_ref[...] = ab
        da2_ref[...] = _lrelu(jnp.dot(ab, xw2_ref[...],
                                      preferred_element_type=jnp.float32))


def _gcn_two_layers(adj, nf, w1, w2, bm=256):
    m = adj.shape[0]
    n = w1.shape[1]
    kin = nf.shape[1]
    ni = m // bm
    last = ni - 1
    return pl.pallas_call(
        functools.partial(_agg2_body, bm=bm),
        grid=(2, ni),
        in_specs=[
            pl.BlockSpec((bm, m), lambda p, i: ((1 - p) * i + p * last, 0)),
            pl.BlockSpec((m, kin), lambda p, i: (0, 0)),
            pl.BlockSpec((kin, n), lambda p, i: (0, 0)),
            pl.BlockSpec((n, n), lambda p, i: (0, 0)),
        ],
        out_specs=(
            pl.BlockSpec((bm, n), lambda p, i: ((1 - p) * i + p * last, 0)),
            pl.BlockSpec((bm, n), lambda p, i: (i * p, 0)),
            pl.BlockSpec((bm, m), lambda p, i: (i * p, 0)),
            pl.BlockSpec((1, bm), lambda p, i: (0, (1 - p) * i + p * last)),
        ),
        out_shape=(
            jax.ShapeDtypeStruct((m, n), jnp.float32),
            jax.ShapeDtypeStruct((m, n), jnp.float32),
            jax.ShapeDtypeStruct((m, m), jnp.bfloat16),
            jax.ShapeDtypeStruct((1, m), jnp.float32),
        ),
        scratch_shapes=[
            pltpu.VMEM((m, m), jnp.bfloat16),
            pltpu.VMEM((m, n), jnp.float32),
            pltpu.VMEM((m, n), jnp.bfloat16),
        ],
        compiler_params=pltpu.CompilerParams(
            dimension_semantics=("arbitrary", "arbitrary")),
    )(adj, nf, w1, w2)


def _qtower_compute(qa_ref, nfq_ref, w1_ref, w2_ref, g_ref, att_ref, emb_ref,
                    *, C):
    qa = qa_ref[...]
    q1 = _lrelu(jnp.dot(qa, jnp.dot(nfq_ref[...], w1_ref[...],
                                    preferred_element_type=jnp.float32),
                        preferred_element_type=jnp.float32))
    g1 = g_ref[:C, :]
    c1 = lax.dot_general(_l2rows(q1), _l2rows(g1), (((1,), (1,)), ((), ())),
                         preferred_element_type=jnp.float32)
    h1 = jnp.dot(c1, g1, preferred_element_type=jnp.float32)
    h1n = h1 / jnp.maximum(
        jnp.sqrt(jnp.sum(h1 * h1, axis=0, keepdims=True)), 1e-12)
    att_q1 = _lrelu(q1 + h1n)
    q2 = _lrelu(jnp.dot(qa, jnp.dot(att_q1, w2_ref[...],
                                    preferred_element_type=jnp.float32),
                        preferred_element_type=jnp.float32))
    g2 = g_ref[C:, :]
    c2 = lax.dot_general(_l2rows(q2), _l2rows(g2), (((1,), (1,)), ((), ())),
                         preferred_element_type=jnp.float32)
    h2 = jnp.dot(c2, g2, preferred_element_type=jnp.float32)
    h2n = h2 / jnp.maximum(
        jnp.sqrt(jnp.sum(h2 * h2, axis=0, keepdims=True)), 1e-12)
    att = _lrelu(q2 + h2n)
    att_ref[...] = att
    emb_ref[...] = jnp.sum(att, axis=0, keepdims=True) / q2.shape[0]


def _maskstats_body(a_ref, da2_ref, diag_ref, qa_ref, nfq_ref, w1q_ref,
                    w2q_ref, g12_ref, thr_ref, att_ref, end_ref, out_ref,
                    attq_ref, fm_ref, fmb_ref, g_ref, msk_ref, emb_ref,
                    *, bm, C):
    p = pl.program_id(0)
    i = pl.program_id(1)

    @pl.when((p == 0) & (i == 0))
    def _():
        _qtower_compute(qa_ref, nfq_ref, w1q_ref, w2q_ref, g12_ref,
                        attq_ref, emb_ref, C=C)

    @pl.when(p == 0)
    def _():
        att = _lrelu(da2_ref[...])
        att_ref[...] = att
        emb = emb_ref[...]
        emb_norm = jnp.sqrt(jnp.sum(emb * emb))
        row_norm = jnp.sqrt(jnp.sum(att * att, axis=1))
        num = jnp.sum(att * emb, axis=1)
        den = jnp.maximum(emb_norm * row_norm, 1e-8)
        endv = num / den
        end_ref[...] = endv[None, :]
        maskv = (endv > thr_ref[0]).astype(jnp.float32)
        fm = att * maskv[:, None]
        fm_ref[pl.ds(i * bm, bm), :] = fm
        fmb_ref[pl.ds(i * bm, bm), :] = fm.astype(jnp.bfloat16)
        msk_ref[pl.ds(i, 1), :] = maskv[None, :]

        @pl.when(i == 0)
        def _():
            g_ref[...] = jnp.zeros_like(g_ref)

        g_ref[...] += lax.dot_general(fm, fm, (((0,), (0,)), ((), ())),
                                      preferred_element_type=jnp.float32)

    @pl.when(p == 1)
    def _():
        a = a_ref[...]
        fmi = fm_ref[pl.ds(i * bm, bm), :]
        tfm = jnp.dot(a, fmb_ref[...], preferred_element_type=jnp.float32)
        fmg = jnp.dot(fmi, g_ref[...], preferred_element_type=jnp.float32)
        qf = jnp.maximum(jnp.sum(fmi * fmg, axis=1), 0.0)
        inv = 1.0 / jnp.maximum(jnp.sqrt(qf), 1e-12)
        rowdot = jnp.sum(fmi * tfm, axis=1)
        total_c = jnp.sum(rowdot * inv)
        fnorm2 = jnp.sum(fmi * fmi, axis=1)
        diag = diag_ref[0, pl.ds(i * bm, bm)]
        tr_c = jnp.sum(fnorm2 * diag * inv)
        cnt_c = jnp.sum(msk_ref[pl.ds(i, 1), :])
        lane = lax.broadcasted_iota(jnp.int32, (1, 1, 128), 2)
        out_ref[...] = jnp.where(
            lane == 0, total_c,
            jnp.where(lane == 1, tr_c, jnp.where(lane == 2, cnt_c, 0.0)))


def _mask_stats_stage(abf, da2, diag, query_adj, nf_q, w1_q, w2_q, g12, thr,
                      bm=1024):
    m, n = da2.shape
    nq = query_adj.shape[0]
    kq = nf_q.shape[1]
    C = g12.shape[0] // 2
    ni = m // bm
    last = ni - 1
    return pl.pallas_call(
        functools.partial(_maskstats_body, bm=bm, C=C),
        grid=(2, ni),
        in_specs=[
            pl.BlockSpec((bm, m), lambda p, i: (i * p, 0)),
            pl.BlockSpec((bm, n), lambda p, i: ((1 - p) * i + p * last, 0)),
            pl.BlockSpec((1, m), lambda p, i: (0, 0)),
            pl.BlockSpec((nq, nq), lambda p, i: (0, 0)),
            pl.BlockSpec((nq, kq), lambda p, i: (0, 0)),
            pl.BlockSpec((kq, n), lambda p, i: (0, 0)),
            pl.BlockSpec((n, n), lambda p, i: (0, 0)),
            pl.BlockSpec((2 * C, n), lambda p, i: (0, 0)),
            pl.BlockSpec(memory_space=pltpu.SMEM),
        ],
        out_specs=(
            pl.BlockSpec((bm, n), lambda p, i: ((1 - p) * i + p * last, 0)),
            pl.BlockSpec((1, bm), lambda p, i: (0, (1 - p) * i + p * last)),
            pl.BlockSpec((1, 1, 128), lambda p, i: (i * p, 0, 0)),
            pl.BlockSpec((nq, n), lambda p, i: (0, 0)),
        ),
        out_shape=(
            jax.ShapeDtypeStruct((m, n), jnp.float32),
            jax.ShapeDtypeStruct((1, m), jnp.float32),
            jax.ShapeDtypeStruct((ni, 1, 128), jnp.float32),
            jax.ShapeDtypeStruct((nq, n), jnp.float32),
        ),
        scratch_shapes=[
            pltpu.VMEM((m, n), jnp.float32),
            pltpu.VMEM((m, n), jnp.bfloat16),
            pltpu.VMEM((n, n), jnp.float32),
            pltpu.VMEM((ni, bm), jnp.float32),
            pltpu.VMEM((1, n), jnp.float32),
        ],
        compiler_params=pltpu.CompilerParams(
            dimension_semantics=("arbitrary", "arbitrary")),
    )(abf, da2, diag, query_adj, nf_q, w1_q, w2_q, g12, thr)


def kernel(target_adj, node_features_da, query_adj, node_features_q,
           candidate_set, candidate_adj, threshold, W1_da, W1_q, W2_da,
           W2_q):
    del candidate_adj

    da1, da2, abf, adiag = _gcn_two_layers(target_adj, node_features_da,
                                           W1_da, W2_da)

    g12 = _sc_gather2(da1, da2, candidate_set)

    thr = jnp.reshape(threshold.astype(jnp.float32), (1,))
    att_da2, end, stats, att_q2 = _mask_stats_stage(
        abf, da2, adiag, query_adj, node_features_q, W1_q, W2_q, g12, thr)

    total = jnp.sum(stats[:, 0, 0])
    tr = jnp.sum(stats[:, 0, 1])
    cnt = jnp.sum(stats[:, 0, 2])
    pre_avg_degree = jnp.where(cnt > 0, total / jnp.maximum(cnt, 1.0), 0.0)
    pre_density = jnp.where(cnt > 0,
                            2.0 * total / (tr * (tr - 1.0) + 1e-4), 0.0)
    pre_avg_nodes = jnp.where(cnt > 0, tr, 0.0)
    return end, att_da2, att_q2, pre_avg_degree, pre_density, pre_avg_nodes


# speedup vs baseline: 1.0534x; 1.0287x over previous
import functools

import jax
import jax.numpy as jnp
from jax import lax
from jax.experimental import pallas as pl
from jax.experimental.pallas import tpu as pltpu
from jax.experimental.pallas import tpu_sc as plsc


def _lrelu(x):
    return jnp.where(x >= 0, x, 0.01 * x)


def _l2rows(x):
    return x / jnp.maximum(jnp.sqrt(jnp.sum(x * x, axis=1, keepdims=True)), 1e-12)


def _sc_gather2(table1, table2, idx):
    C = idx.shape[0]
    D = table1.shape[1]
    info = plsc.get_sparse_core_info()
    nw = info.num_cores * info.num_subcores
    half = nw // 2
    b = C // half
    mesh = plsc.VectorSubcoreMesh(core_axis_name="c", subcore_axis_name="s")

    @functools.partial(
        pl.kernel,
        mesh=mesh,
        out_type=jax.ShapeDtypeStruct((2 * C, D), jnp.float32),
        scratch_types=[
            pltpu.VMEM((b,), jnp.int32),
            pltpu.VMEM((b, D), jnp.float32),
            pltpu.SemaphoreType.DMA,
        ],
    )
    def k(t1_hbm, t2_hbm, idx_hbm, out_hbm, idx_v, rows_v, sem):
        wid = lax.axis_index("s") * info.num_cores + lax.axis_index("c")
        slot = wid % half
        base = slot * b
        pltpu.sync_copy(idx_hbm.at[pl.ds(base, b)], idx_v)

        @pl.when(wid < half)
        def _():
            pltpu.async_copy(t1_hbm.at[idx_v], rows_v, sem).wait()
            pltpu.sync_copy(rows_v, out_hbm.at[pl.ds(base, b)])

        @pl.when(wid >= half)
        def _():
            pltpu.async_copy(t2_hbm.at[idx_v], rows_v, sem).wait()
            pltpu.sync_copy(rows_v, out_hbm.at[pl.ds(C + base, b)])

    return k(table1, table2, idx)


def _agg2_body(a_ref, nf_ref, w1_ref, w2_ref, da1_ref, da2_ref, abfo_ref,
               abf_ref, xw1_ref, xw2_ref, *, bm):
    p = pl.program_id(0)
    i = pl.program_id(1)

    @pl.when((p == 0) & (i == 0))
    def _():
        xw1_ref[...] = jnp.dot(nf_ref[...], w1_ref[...],
                               preferred_element_type=jnp.float32)

    @pl.when(p == 0)
    def _():
        a = a_ref[...]
        abf_ref[pl.ds(i * bm, bm), :] = a.astype(jnp.bfloat16)
        da1 = _lrelu(jnp.dot(a, xw1_ref[...],
                             preferred_element_type=jnp.float32))
        da1_ref[...] = da1
        xw2_ref[pl.ds(i * bm, bm), :] = jnp.dot(
            _lrelu(da1), w2_ref[...],
            preferred_element_type=jnp.float32).astype(jnp.bfloat16)

    @pl.when(p == 1)
    def _():
        ab = abf_ref[pl.ds(i * bm, bm), :]
        abfo_ref[...] = ab
        da2_ref[...] = _lrelu(jnp.dot(ab, xw2_ref[...],
                                      preferred_element_type=jnp.float32))


def _gcn_two_layers(adj, nf, w1, w2, bm=256):
    m = adj.shape[0]
    n = w1.shape[1]
    kin = nf.shape[1]
    ni = m // bm
    last = ni - 1
    return pl.pallas_call(
        functools.partial(_agg2_body, bm=bm),
        grid=(2, ni),
        in_specs=[
            pl.BlockSpec((bm, m), lambda p, i: ((1 - p) * i + p * last, 0)),
            pl.BlockSpec((m, kin), lambda p, i: (0, 0)),
            pl.BlockSpec((kin, n), lambda p, i: (0, 0)),
            pl.BlockSpec((n, n), lambda p, i: (0, 0)),
        ],
        out_specs=(
            pl.BlockSpec((bm, n), lambda p, i: ((1 - p) * i + p * last, 0)),
            pl.BlockSpec((bm, n), lambda p, i: (i * p, 0)),
            pl.BlockSpec((bm, m), lambda p, i: (i * p, 0)),
        ),
        out_shape=(
            jax.ShapeDtypeStruct((m, n), jnp.float32),
            jax.ShapeDtypeStruct((m, n), jnp.float32),
            jax.ShapeDtypeStruct((m, m), jnp.bfloat16),
        ),
        scratch_shapes=[
            pltpu.VMEM((m, m), jnp.bfloat16),
            pltpu.VMEM((m, n), jnp.float32),
            pltpu.VMEM((m, n), jnp.bfloat16),
        ],
        compiler_params=pltpu.CompilerParams(
            dimension_semantics=("arbitrary", "arbitrary")),
    )(adj, nf, w1, w2)


def _qtower_compute(qa_ref, nfq_ref, w1_ref, w2_ref, g_ref, att_ref, emb_ref,
                    *, C):
    qa = qa_ref[...]
    q1 = _lrelu(jnp.dot(qa, jnp.dot(nfq_ref[...], w1_ref[...],
                                    preferred_element_type=jnp.float32),
                        preferred_element_type=jnp.float32))
    g1 = g_ref[:C, :]
    c1 = lax.dot_general(_l2rows(q1), _l2rows(g1), (((1,), (1,)), ((), ())),
                         preferred_element_type=jnp.float32)
    h1 = jnp.dot(c1, g1, preferred_element_type=jnp.float32)
    h1n = h1 / jnp.maximum(
        jnp.sqrt(jnp.sum(h1 * h1, axis=0, keepdims=True)), 1e-12)
    att_q1 = _lrelu(q1 + h1n)
    q2 = _lrelu(jnp.dot(qa, jnp.dot(att_q1, w2_ref[...],
                                    preferred_element_type=jnp.float32),
                        preferred_element_type=jnp.float32))
    g2 = g_ref[C:, :]
    c2 = lax.dot_general(_l2rows(q2), _l2rows(g2), (((1,), (1,)), ((), ())),
                         preferred_element_type=jnp.float32)
    h2 = jnp.dot(c2, g2, preferred_element_type=jnp.float32)
    h2n = h2 / jnp.maximum(
        jnp.sqrt(jnp.sum(h2 * h2, axis=0, keepdims=True)), 1e-12)
    att = _lrelu(q2 + h2n)
    att_ref[...] = att
    emb_ref[...] = jnp.sum(att, axis=0, keepdims=True) / q2.shape[0]


def _maskstats_body(a_ref, da2_ref, qa_ref, nfq_ref, w1q_ref,
                    w2q_ref, g12_ref, thr_ref, att_ref, end_ref, out_ref,
                    attq_ref, fm_ref, fmb_ref, g_ref, msk_ref, emb_ref,
                    *, bm, C):
    p = pl.program_id(0)
    i = pl.program_id(1)

    @pl.when((p == 0) & (i == 0))
    def _():
        _qtower_compute(qa_ref, nfq_ref, w1q_ref, w2q_ref, g12_ref,
                        attq_ref, emb_ref, C=C)

    @pl.when(p == 0)
    def _():
        att = _lrelu(da2_ref[...])
        att_ref[...] = att
        emb = emb_ref[...]
        emb_norm = jnp.sqrt(jnp.sum(emb * emb))
        row_norm = jnp.sqrt(jnp.sum(att * att, axis=1))
        num = jnp.sum(att * emb, axis=1)
        den = jnp.maximum(emb_norm * row_norm, 1e-8)
        endv = num / den
        end_ref[...] = endv[None, :]
        maskv = (endv > thr_ref[0]).astype(jnp.float32)
        fm = att * maskv[:, None]
        fm_ref[pl.ds(i * bm, bm), :] = fm
        fmb_ref[pl.ds(i * bm, bm), :] = fm.astype(jnp.bfloat16)
        msk_ref[pl.ds(i, 1), :] = maskv[None, :]

        @pl.when(i == 0)
        def _():
            g_ref[...] = jnp.zeros_like(g_ref)

        g_ref[...] += lax.dot_general(fm, fm, (((0,), (0,)), ((), ())),
                                      preferred_element_type=jnp.float32)

    @pl.when(p == 1)
    def _():
        a = a_ref[...]
        fmi = fm_ref[pl.ds(i * bm, bm), :]
        tfm = jnp.dot(a, fmb_ref[...], preferred_element_type=jnp.float32)
        fmg = jnp.dot(fmi, g_ref[...], preferred_element_type=jnp.float32)
        qf = jnp.maximum(jnp.sum(fmi * fmg, axis=1), 0.0)
        inv = 1.0 / jnp.maximum(jnp.sqrt(qf), 1e-12)
        rowdot = jnp.sum(fmi * tfm, axis=1)
        total_c = jnp.sum(rowdot * inv)
        fnorm2 = jnp.sum(fmi * fmi, axis=1)
        sub = a_ref[:, pl.ds(i * bm, bm)]
        r = lax.broadcasted_iota(jnp.int32, (bm, bm), 0)
        c = lax.broadcasted_iota(jnp.int32, (bm, bm), 1)
        diag = jnp.sum(jnp.where(c == r, sub, 0).astype(jnp.float32), axis=1)
        tr_c = jnp.sum(fnorm2 * diag * inv)
        cnt_c = jnp.sum(msk_ref[pl.ds(i, 1), :])
        lane = lax.broadcasted_iota(jnp.int32, (1, 1, 128), 2)
        out_ref[...] = jnp.where(
            lane == 0, total_c,
            jnp.where(lane == 1, tr_c, jnp.where(lane == 2, cnt_c, 0.0)))


def _mask_stats_stage(abf, da2, query_adj, nf_q, w1_q, w2_q, g12, thr,
                      bm=1024):
    m, n = da2.shape
    nq = query_adj.shape[0]
    kq = nf_q.shape[1]
    C = g12.shape[0] // 2
    ni = m // bm
    last = ni - 1
    return pl.pallas_call(
        functools.partial(_maskstats_body, bm=bm, C=C),
        grid=(2, ni),
        in_specs=[
            pl.BlockSpec((bm, m), lambda p, i: (i * p, 0)),
            pl.BlockSpec((bm, n), lambda p, i: ((1 - p) * i + p * last, 0)),
            pl.BlockSpec((nq, nq), lambda p, i: (0, 0)),
            pl.BlockSpec((nq, kq), lambda p, i: (0, 0)),
            pl.BlockSpec((kq, n), lambda p, i: (0, 0)),
            pl.BlockSpec((n, n), lambda p, i: (0, 0)),
            pl.BlockSpec((2 * C, n), lambda p, i: (0, 0)),
            pl.BlockSpec(memory_space=pltpu.SMEM),
        ],
        out_specs=(
            pl.BlockSpec((bm, n), lambda p, i: ((1 - p) * i + p * last, 0)),
            pl.BlockSpec((1, bm), lambda p, i: (0, (1 - p) * i + p * last)),
            pl.BlockSpec((1, 1, 128), lambda p, i: (i * p, 0, 0)),
            pl.BlockSpec((nq, n), lambda p, i: (0, 0)),
        ),
        out_shape=(
            jax.ShapeDtypeStruct((m, n), jnp.float32),
            jax.ShapeDtypeStruct((1, m), jnp.float32),
            jax.ShapeDtypeStruct((ni, 1, 128), jnp.float32),
            jax.ShapeDtypeStruct((nq, n), jnp.float32),
        ),
        scratch_shapes=[
            pltpu.VMEM((m, n), jnp.float32),
            pltpu.VMEM((m, n), jnp.bfloat16),
            pltpu.VMEM((n, n), jnp.float32),
            pltpu.VMEM((ni, bm), jnp.float32),
            pltpu.VMEM((1, n), jnp.float32),
        ],
        compiler_params=pltpu.CompilerParams(
            dimension_semantics=("arbitrary", "arbitrary")),
    )(abf, da2, query_adj, nf_q, w1_q, w2_q, g12, thr)


def kernel(target_adj, node_features_da, query_adj, node_features_q,
           candidate_set, candidate_adj, threshold, W1_da, W1_q, W2_da,
           W2_q):
    del candidate_adj

    da1, da2, abf = _gcn_two_layers(target_adj, node_features_da, W1_da,
                                    W2_da)

    g12 = _sc_gather2(da1, da2, candidate_set)

    thr = jnp.reshape(threshold.astype(jnp.float32), (1,))
    att_da2, end, stats, att_q2 = _mask_stats_stage(
        abf, da2, query_adj, node_features_q, W1_q, W2_q, g12, thr)

    total = jnp.sum(stats[:, 0, 0])
    tr = jnp.sum(stats[:, 0, 1])
    cnt = jnp.sum(stats[:, 0, 2])
    pre_avg_degree = jnp.where(cnt > 0, total / jnp.maximum(cnt, 1.0), 0.0)
    pre_density = jnp.where(cnt > 0,
                            2.0 * total / (tr * (tr - 1.0) + 1e-4), 0.0)
    pre_avg_nodes = jnp.where(cnt > 0, tr, 0.0)
    return end, att_da2, att_q2, pre_avg_degree, pre_density, pre_avg_nodes


# speedup vs baseline: 1.0584x; 1.0047x over previous
import functools

import jax
import jax.numpy as jnp
from jax import lax
from jax.experimental import pallas as pl
from jax.experimental.pallas import tpu as pltpu
from jax.experimental.pallas import tpu_sc as plsc


def _lrelu(x):
    return jnp.where(x >= 0, x, 0.01 * x)


def _l2rows(x):
    return x / jnp.maximum(jnp.sqrt(jnp.sum(x * x, axis=1, keepdims=True)), 1e-12)


def _sc_gather2(table1, table2, idx):
    C = idx.shape[0]
    D = table1.shape[1]
    info = plsc.get_sparse_core_info()
    nw = info.num_cores * info.num_subcores
    half = nw // 2
    b = C // half
    mesh = plsc.VectorSubcoreMesh(core_axis_name="c", subcore_axis_name="s")

    @functools.partial(
        pl.kernel,
        mesh=mesh,
        out_type=jax.ShapeDtypeStruct((2 * C, D), jnp.float32),
        scratch_types=[
            pltpu.VMEM((b,), jnp.int32),
            pltpu.VMEM((b, D), jnp.float32),
            pltpu.SemaphoreType.DMA,
        ],
    )
    def k(t1_hbm, t2_hbm, idx_hbm, out_hbm, idx_v, rows_v, sem):
        wid = lax.axis_index("s") * info.num_cores + lax.axis_index("c")
        slot = wid % half
        base = slot * b
        pltpu.sync_copy(idx_hbm.at[pl.ds(base, b)], idx_v)

        @pl.when(wid < half)
        def _():
            pltpu.async_copy(t1_hbm.at[idx_v], rows_v, sem).wait()
            pltpu.sync_copy(rows_v, out_hbm.at[pl.ds(base, b)])

        @pl.when(wid >= half)
        def _():
            pltpu.async_copy(t2_hbm.at[idx_v], rows_v, sem).wait()
            pltpu.sync_copy(rows_v, out_hbm.at[pl.ds(C + base, b)])

    return k(table1, table2, idx)


def _agg2_body(a_ref, nf_ref, w1_ref, w2_ref, da1_ref, da2_ref, abfo_ref,
               abf_ref, xw1_ref, xw2_ref, *, bm):
    p = pl.program_id(0)
    i = pl.program_id(1)

    @pl.when((p == 0) & (i == 0))
    def _():
        xw1_ref[...] = jnp.dot(nf_ref[...], w1_ref[...],
                               preferred_element_type=jnp.float32
                               ).astype(jnp.bfloat16)

    @pl.when(p == 0)
    def _():
        ab = a_ref[...].astype(jnp.bfloat16)
        abf_ref[pl.ds(i * bm, bm), :] = ab
        da1 = _lrelu(jnp.dot(ab, xw1_ref[...],
                             preferred_element_type=jnp.float32))
        da1_ref[...] = da1
        xw2_ref[pl.ds(i * bm, bm), :] = jnp.dot(
            _lrelu(da1), w2_ref[...],
            preferred_element_type=jnp.float32).astype(jnp.bfloat16)

    @pl.when(p == 1)
    def _():
        ab = abf_ref[pl.ds(i * bm, bm), :]
        abfo_ref[...] = ab
        da2_ref[...] = _lrelu(jnp.dot(ab, xw2_ref[...],
                                      preferred_element_type=jnp.float32))


def _gcn_two_layers(adj, nf, w1, w2, bm=256):
    m = adj.shape[0]
    n = w1.shape[1]
    kin = nf.shape[1]
    ni = m // bm
    last = ni - 1
    return pl.pallas_call(
        functools.partial(_agg2_body, bm=bm),
        grid=(2, ni),
        in_specs=[
            pl.BlockSpec((bm, m), lambda p, i: ((1 - p) * i + p * last, 0)),
            pl.BlockSpec((m, kin), lambda p, i: (0, 0)),
            pl.BlockSpec((kin, n), lambda p, i: (0, 0)),
            pl.BlockSpec((n, n), lambda p, i: (0, 0)),
        ],
        out_specs=(
            pl.BlockSpec((bm, n), lambda p, i: ((1 - p) * i + p * last, 0)),
            pl.BlockSpec((bm, n), lambda p, i: (i * p, 0)),
            pl.BlockSpec((bm, m), lambda p, i: (i * p, 0)),
        ),
        out_shape=(
            jax.ShapeDtypeStruct((m, n), jnp.float32),
            jax.ShapeDtypeStruct((m, n), jnp.float32),
            jax.ShapeDtypeStruct((m, m), jnp.bfloat16),
        ),
        scratch_shapes=[
            pltpu.VMEM((m, m), jnp.bfloat16),
            pltpu.VMEM((m, n), jnp.bfloat16),
            pltpu.VMEM((m, n), jnp.bfloat16),
        ],
        compiler_params=pltpu.CompilerParams(
            dimension_semantics=("arbitrary", "arbitrary")),
    )(adj, nf, w1, w2)


def _qtower_compute(qa_ref, nfq_ref, w1_ref, w2_ref, g_ref, att_ref, emb_ref,
                    *, C):
    qa = qa_ref[...]
    q1 = _lrelu(jnp.dot(qa, jnp.dot(nfq_ref[...], w1_ref[...],
                                    preferred_element_type=jnp.float32),
                        preferred_element_type=jnp.float32))
    g1 = g_ref[:C, :]
    c1 = lax.dot_general(_l2rows(q1), _l2rows(g1), (((1,), (1,)), ((), ())),
                         preferred_element_type=jnp.float32)
    h1 = jnp.dot(c1, g1, preferred_element_type=jnp.float32)
    h1n = h1 / jnp.maximum(
        jnp.sqrt(jnp.sum(h1 * h1, axis=0, keepdims=True)), 1e-12)
    att_q1 = _lrelu(q1 + h1n)
    q2 = _lrelu(jnp.dot(qa, jnp.dot(att_q1, w2_ref[...],
                                    preferred_element_type=jnp.float32),
                        preferred_element_type=jnp.float32))
    g2 = g_ref[C:, :]
    c2 = lax.dot_general(_l2rows(q2), _l2rows(g2), (((1,), (1,)), ((), ())),
                         preferred_element_type=jnp.float32)
    h2 = jnp.dot(c2, g2, preferred_element_type=jnp.float32)
    h2n = h2 / jnp.maximum(
        jnp.sqrt(jnp.sum(h2 * h2, axis=0, keepdims=True)), 1e-12)
    att = _lrelu(q2 + h2n)
    att_ref[...] = att
    emb_ref[...] = jnp.sum(att, axis=0, keepdims=True) / q2.shape[0]


def _maskstats_body(a_ref, da2_ref, qa_ref, nfq_ref, w1q_ref,
                    w2q_ref, g12_ref, thr_ref, att_ref, end_ref, out_ref,
                    attq_ref, fm_ref, fmb_ref, g_ref, msk_ref, emb_ref,
                    *, bm, C):
    p = pl.program_id(0)
    i = pl.program_id(1)

    @pl.when((p == 0) & (i == 0))
    def _():
        _qtower_compute(qa_ref, nfq_ref, w1q_ref, w2q_ref, g12_ref,
                        attq_ref, emb_ref, C=C)

    @pl.when(p == 0)
    def _():
        att = _lrelu(da2_ref[...])
        att_ref[...] = att
        emb = emb_ref[...]
        emb_norm = jnp.sqrt(jnp.sum(emb * emb))
        row_norm = jnp.sqrt(jnp.sum(att * att, axis=1))
        num = jnp.sum(att * emb, axis=1)
        den = jnp.maximum(emb_norm * row_norm, 1e-8)
        endv = num / den
        end_ref[...] = endv[None, :]
        maskv = (endv > thr_ref[0]).astype(jnp.float32)
        fm = att * maskv[:, None]
        fm_ref[pl.ds(i * bm, bm), :] = fm
        fmb_ref[pl.ds(i * bm, bm), :] = fm.astype(jnp.bfloat16)
        msk_ref[pl.ds(i, 1), :] = maskv[None, :]

        @pl.when(i == 0)
        def _():
            g_ref[...] = jnp.zeros_like(g_ref)

        g_ref[...] += lax.dot_general(fm, fm, (((0,), (0,)), ((), ())),
                                      preferred_element_type=jnp.float32)

    @pl.when(p == 1)
    def _():
        a = a_ref[...]
        fmi = fm_ref[pl.ds(i * bm, bm), :]
        tfm = jnp.dot(a, fmb_ref[...], preferred_element_type=jnp.float32)
        fmg = jnp.dot(fmi, g_ref[...], preferred_element_type=jnp.float32)
        qf = jnp.maximum(jnp.sum(fmi * fmg, axis=1), 0.0)
        inv = 1.0 / jnp.maximum(jnp.sqrt(qf), 1e-12)
        rowdot = jnp.sum(fmi * tfm, axis=1)
        total_c = jnp.sum(rowdot * inv)
        fnorm2 = jnp.sum(fmi * fmi, axis=1)
        sub = a_ref[:, pl.ds(i * bm, bm)]
        r = lax.broadcasted_iota(jnp.int32, (bm, bm), 0)
        c = lax.broadcasted_iota(jnp.int32, (bm, bm), 1)
        diag = jnp.sum(jnp.where(c == r, sub, 0).astype(jnp.float32), axis=1)
        tr_c = jnp.sum(fnorm2 * diag * inv)
        cnt_c = jnp.sum(msk_ref[pl.ds(i, 1), :])
        lane = lax.broadcasted_iota(jnp.int32, (1, 1, 128), 2)
        out_ref[...] = jnp.where(
            lane == 0, total_c,
            jnp.where(lane == 1, tr_c, jnp.where(lane == 2, cnt_c, 0.0)))


def _mask_stats_stage(abf, da2, query_adj, nf_q, w1_q, w2_q, g12, thr,
                      bm=1024):
    m, n = da2.shape
    nq = query_adj.shape[0]
    kq = nf_q.shape[1]
    C = g12.shape[0] // 2
    ni = m // bm
    last = ni - 1
    return pl.pallas_call(
        functools.partial(_maskstats_body, bm=bm, C=C),
        grid=(2, ni),
        in_specs=[
            pl.BlockSpec((bm, m), lambda p, i: (i * p, 0)),
            pl.BlockSpec((bm, n), lambda p, i: ((1 - p) * i + p * last, 0)),
            pl.BlockSpec((nq, nq), lambda p, i: (0, 0)),
            pl.BlockSpec((nq, kq), lambda p, i: (0, 0)),
            pl.BlockSpec((kq, n), lambda p, i: (0, 0)),
            pl.BlockSpec((n, n), lambda p, i: (0, 0)),
            pl.BlockSpec((2 * C, n), lambda p, i: (0, 0)),
            pl.BlockSpec(memory_space=pltpu.SMEM),
        ],
        out_specs=(
            pl.BlockSpec((bm, n), lambda p, i: ((1 - p) * i + p * last, 0)),
            pl.BlockSpec((1, bm), lambda p, i: (0, (1 - p) * i + p * last)),
            pl.BlockSpec((1, 1, 128), lambda p, i: (i * p, 0, 0)),
            pl.BlockSpec((nq, n), lambda p, i: (0, 0)),
        ),
        out_shape=(
            jax.ShapeDtypeStruct((m, n), jnp.float32),
            jax.ShapeDtypeStruct((1, m), jnp.float32),
            jax.ShapeDtypeStruct((ni, 1, 128), jnp.float32),
            jax.ShapeDtypeStruct((nq, n), jnp.float32),
        ),
        scratch_shapes=[
            pltpu.VMEM((m, n), jnp.float32),
            pltpu.VMEM((m, n), jnp.bfloat16),
            pltpu.VMEM((n, n), jnp.float32),
            pltpu.VMEM((ni, bm), jnp.float32),
            pltpu.VMEM((1, n), jnp.float32),
        ],
        compiler_params=pltpu.CompilerParams(
            dimension_semantics=("arbitrary", "arbitrary")),
    )(abf, da2, query_adj, nf_q, w1_q, w2_q, g12, thr)


def kernel(target_adj, node_features_da, query_adj, node_features_q,
           candidate_set, candidate_adj, threshold, W1_da, W1_q, W2_da,
           W2_q):
    del candidate_adj

    da1, da2, abf = _gcn_two_layers(target_adj, node_features_da, W1_da,
                                    W2_da)

    g12 = _sc_gather2(da1, da2, candidate_set)

    thr = jnp.reshape(threshold.astype(jnp.float32), (1,))
    att_da2, end, stats, att_q2 = _mask_stats_stage(
        abf, da2, query_adj, node_features_q, W1_q, W2_q, g12, thr)

    total = jnp.sum(stats[:, 0, 0])
    tr = jnp.sum(stats[:, 0, 1])
    cnt = jnp.sum(stats[:, 0, 2])
    pre_avg_degree = jnp.where(cnt > 0, total / jnp.maximum(cnt, 1.0), 0.0)
    pre_density = jnp.where(cnt > 0,
                            2.0 * total / (tr * (tr - 1.0) + 1e-4), 0.0)
    pre_avg_nodes = jnp.where(cnt > 0, tr, 0.0)
    return end, att_da2, att_q2, pre_avg_degree, pre_density, pre_avg_nodes
